# baseline (speedup 1.0000x reference)
import functools

import jax
import jax.numpy as jnp
from jax import lax
from jax.experimental import pallas as pl
from jax.experimental.pallas import tpu as pltpu
from jax.experimental.pallas import tpu_sc as plsc

NN = 100000
EE = 100000
DD = 128
NT = 32
PT = 3136
NP = NT * PT
CH = 448
NCH = PT // CH
CW_CH = 3584
EPAD = NP - EE
BN_EPS = 1e-5
EPS = 1e-5

_f32 = jnp.float32
_mesh = plsc.VectorSubcoreMesh(core_axis_name="c", subcore_axis_name="s")
_sc_params = pltpu.CompilerParams(needs_layout_passes=False)


def _wid():
    return lax.axis_index("s") * 2 + lax.axis_index("c")


def _k1_body(h_ref, e_ref, wr_ref, br_ref, wc_ref, bc_ref, wv_ref, vb_ref,
             xr_ref, xc_ref, hv_ref, ep_ref):
    i = pl.program_id(0)
    rows = i * 512 + lax.broadcasted_iota(jnp.int32, (512, 1), 0)
    m = rows < NN
    h_raw = h_ref[...]
    e_raw = e_ref[...]
    hb = jnp.where(m, h_raw, 0.0)
    eb = jnp.where(m, e_raw, 0.0)
    he = jnp.concatenate([hb, eb], axis=1)
    xr_ref[...] = jnp.dot(he, wr_ref[...], preferred_element_type=_f32) + br_ref[...]
    xc_ref[...] = jnp.where(
        m, jnp.dot(he, wc_ref[...], preferred_element_type=_f32) + bc_ref[...], -1e30)
    hv_ref[...] = jnp.dot(hb, wv_ref[...], preferred_element_type=_f32) + vb_ref[...]
    ep_ref[...] = jnp.where(m, e_raw, -1e30)


def _k1(h, e, wr, br, wc, bc, wv, vb):
    blk = lambda s: pl.BlockSpec(s, lambda i: (0,) * len(s))
    return pl.pallas_call(
        _k1_body,
        grid=(NP // 512,),
        in_specs=[
            pl.BlockSpec((512, DD), lambda i: (i, 0)),
            pl.BlockSpec((512, DD), lambda i: (i, 0)),
            blk((2 * DD, DD)), blk((1, DD)),
            blk((2 * DD, DD)), blk((1, DD)),
            blk((DD, DD)), blk((1, DD)),
        ],
        out_specs=[pl.BlockSpec((512, DD), lambda i: (i, 0))] * 4,
        out_shape=[jax.ShapeDtypeStruct((NP, DD), _f32)] * 4,
    )(h, e, wr, br, wc, bc, wv, vb)


@functools.partial(
    pl.kernel,
    out_type=jax.ShapeDtypeStruct((NP,), jnp.int32),
    mesh=_mesh,
    compiler_params=_sc_params,
    scratch_types=[
        pltpu.VMEM((PT,), jnp.int32),
        pltpu.VMEM((CW_CH,), jnp.int32),
        pltpu.VMEM((CW_CH,), jnp.int32),
    ],
)
def _kw(rowp_hbm, colp_hbm, cw_hbm, slab, rbuf, cbuf):
    wid = _wid()
    base = wid * PT
    neg1 = jnp.full((16,), -1, jnp.int32)
    iota = lax.iota(jnp.int32, 16)

    def init(i, _):
        slab[pl.ds(i * 16, 16)] = neg1
        return 0
    lax.fori_loop(0, PT // 16, init, 0)

    def chunk(cix, _):
        pltpu.sync_copy(rowp_hbm.at[pl.ds(cix * CW_CH, CW_CH)], rbuf)
        pltpu.sync_copy(colp_hbm.at[pl.ds(cix * CW_CH, CW_CH)], cbuf)
        kbase = cix * CW_CH

        def vreg(i, _):
            rv = rbuf[pl.ds(i * 16, 16)]
            cv = cbuf[pl.ds(i * 16, 16)]
            kvec = kbase + i * 16 + iota
            m = (rv >= base) & (rv < base + PT) & (kvec < EE)
            idx = jnp.clip(rv - base, 0, PT - 1)
            plsc.store_scatter(slab, [idx], cv, mask=m)
            return 0
        lax.fori_loop(0, CW_CH // 16, vreg, 0)
        return 0
    lax.fori_loop(0, NP // CW_CH, chunk, 0)
    pltpu.sync_copy(slab, cw_hbm.at[pl.ds(base, PT)])


@functools.partial(
    pl.kernel,
    out_type=jax.ShapeDtypeStruct((NT * 2 * DD,), _f32),
    mesh=_mesh,
    compiler_params=_sc_params,
    scratch_types=[
        pltpu.VMEM((CH,), jnp.int32),
        pltpu.VMEM((CH,), jnp.int32),
        pltpu.VMEM((CH, DD), _f32),
        pltpu.VMEM((CH, DD), _f32),
        pltpu.VMEM((2 * DD,), _f32),
        pltpu.SemaphoreType.DMA,
        pltpu.SemaphoreType.DMA,
    ],
)
def _k2(rowp_hbm, colp_hbm, xr_hbm, xc_hbm, st_hbm,
        ribuf, cibuf, xrb, xcb, outb, sem1, sem2):
    wid = _wid()
    base = wid * PT
    zero = jnp.zeros((16,), _f32)

    def chunk(cix, carry):
        off = base + cix * CH
        pltpu.sync_copy(rowp_hbm.at[pl.ds(off, CH)], ribuf)
        pltpu.sync_copy(colp_hbm.at[pl.ds(off, CH)], cibuf)
        cp1 = pltpu.async_copy(xr_hbm.at[ribuf], xrb, sem1)
        cp2 = pltpu.async_copy(xc_hbm.at[cibuf], xcb, sem2)
        cp1.wait()
        cp2.wait()

        def edge(i, car):
            s = list(car[0])
            q = list(car[1])
            for j in range(8):
                sl = pl.ds(j * 16, 16)
                t = xrb[i, sl] + xcb[i, sl]
                s[j] = s[j] + t
                q[j] = q[j] + t * t
            return (tuple(s), tuple(q))
        return lax.fori_loop(0, CH, edge, carry)

    init = (tuple(zero for _ in range(8)), tuple(zero for _ in range(8)))
    s, q = lax.fori_loop(0, NCH, chunk, init)
    for j in range(8):
        outb[pl.ds(j * 16, 16)] = s[j]
        outb[pl.ds(DD + j * 16, 16)] = q[j]
    pltpu.sync_copy(outb, st_hbm.at[pl.ds(wid * 2 * DD, 2 * DD)])


@functools.partial(
    pl.kernel,
    out_type=(jax.ShapeDtypeStruct((NP, DD), _f32),
              jax.ShapeDtypeStruct((NP, DD), _f32)),
    mesh=_mesh,
    compiler_params=_sc_params,
    scratch_types=[
        pltpu.VMEM((CH,), jnp.int32),
        pltpu.VMEM((CH,), jnp.int32),
        pltpu.VMEM((CH,), jnp.int32),
        pltpu.VMEM((CH, DD), _f32),
        pltpu.VMEM((CH, DD), _f32),
        pltpu.SemaphoreType.DMA,
        pltpu.SemaphoreType.DMA,
    ],
)
def _k3a(cw_hbm, colp_hbm, xc_hbm, hv_hbm, gc_hbm, gv_hbm,
         cwb, colb, idxb, xcgb, gvb, sem1, sem2):
    wid = _wid()
    base = wid * PT

    def chunk(cix, _):
        noff = base + cix * CH
        pltpu.sync_copy(cw_hbm.at[pl.ds(noff, CH)], cwb)
        pltpu.sync_copy(colp_hbm.at[pl.ds(noff, CH)], colb)

        def mk(i, _):
            cwv = cwb[pl.ds(i * 16, 16)]
            idxb[pl.ds(i * 16, 16)] = jnp.where(cwv < 0, NN, cwv)
            return 0
        lax.fori_loop(0, CH // 16, mk, 0)
        cp1 = pltpu.async_copy(xc_hbm.at[idxb], xcgb, sem1)
        cp2 = pltpu.async_copy(hv_hbm.at[colb], gvb, sem2)
        cp1.wait()
        cp2.wait()
        pltpu.sync_copy(xcgb, gc_hbm.at[pl.ds(noff, CH)])
        pltpu.sync_copy(gvb, gv_hbm.at[pl.ds(noff, CH)])
        return 0
    lax.fori_loop(0, NCH, chunk, 0)


def _k3b_body(xr_ref, gc_ref, gv_ref, ep_ref, st_ref, x0_ref, c0_ref,
              g_ref, b_ref, sig_ref, cn_ref):
    i = pl.program_id(0)
    stf = st_ref[...]
    ssum = jnp.sum(stf[:, 0, :], axis=0)
    qsum = jnp.sum(stf[:, 1, :], axis=0)
    t0 = (x0_ref[...] + c0_ref[...])[0]
    ssum = ssum - float(EPAD) * t0
    qsum = qsum - float(EPAD) * (t0 * t0)
    mean = ssum * (1.0 / EE)
    var = jnp.maximum(qsum * (1.0 / EE) - mean * mean, 0.0)
    s = g_ref[...][0] * lax.rsqrt(var + BN_EPS)
    c = b_ref[...][0] - mean * s
    t = xr_ref[...] + gc_ref[...]
    add = jnp.maximum(t * s[None, :] + c[None, :], 0.0)
    pre = ep_ref[...] + add
    sig = jax.nn.sigmoid(pre)
    sig_ref[...] = sig
    ps = jnp.sum(sig, axis=0)
    pn = jnp.sum(sig * gv_ref[...], axis=0)
    acc = jnp.stack([ps, pn], axis=0)

    @pl.when(i == 0)
    def _():
        cn_ref[...] = acc

    @pl.when(i > 0)
    def _():
        cn_ref[...] += acc


def _k3b(xr, gc, gv, ep, st, x0, c0, g, b):
    blk = lambda s: pl.BlockSpec(s, lambda i: (0,) * len(s))
    return pl.pallas_call(
        _k3b_body,
        grid=(NP // 512,),
        in_specs=[
            pl.BlockSpec((512, DD), lambda i: (i, 0)),
            pl.BlockSpec((512, DD), lambda i: (i, 0)),
            pl.BlockSpec((512, DD), lambda i: (i, 0)),
            pl.BlockSpec((512, DD), lambda i: (i, 0)),
            blk((NT, 2, DD)), blk((1, DD)), blk((1, DD)),
            blk((1, DD)), blk((1, DD)),
        ],
        out_specs=[pl.BlockSpec((512, DD), lambda i: (i, 0)),
                   pl.BlockSpec((2, DD), lambda i: (0, 0))],
        out_shape=[jax.ShapeDtypeStruct((NP, DD), _f32),
                   jax.ShapeDtypeStruct((2, DD), _f32)],
    )(xr, gc, gv, ep, st, x0, c0, g, b)


def _k4_body(h_ref, sig_ref, cn_ref, wu_ref, ub_ref, hout_ref, enew_ref):
    cn = cn_ref[...]
    r = 1.0 / (cn[0] + EPS)
    enew_ref[...] = sig_ref[...] * r[None, :]
    hu = jnp.dot(h_ref[...], wu_ref[...], preferred_element_type=_f32) + ub_ref[...]
    hout_ref[...] = jnp.maximum(hu + (cn[1] * r)[None, :], 0.0)


def _k4(h, sig, cn, wu, ub):
    return pl.pallas_call(
        _k4_body,
        grid=(NN // 1000,),
        in_specs=[
            pl.BlockSpec((1000, DD), lambda i: (i, 0)),
            pl.BlockSpec((1000, DD), lambda i: (i, 0)),
            pl.BlockSpec((2, DD), lambda i: (0, 0)),
            pl.BlockSpec((DD, DD), lambda i: (0, 0)),
            pl.BlockSpec((1, DD), lambda i: (0, 0)),
        ],
        out_specs=[pl.BlockSpec((1000, DD), lambda i: (i, 0))] * 2,
        out_shape=[jax.ShapeDtypeStruct((NN, DD), _f32)] * 2,
    )(h, sig, cn, wu, ub)


def kernel(h, e, edge_index, A_w, A_b, B_w, B_b, C_w, C_b, Dm_w, Dm_b,
           U_w, U_b, V_w, V_b, bn_g, bn_b):
    row = edge_index[0]
    col = edge_index[1]
    pad = jnp.zeros((NP - EE,), jnp.int32)
    rowp = jnp.concatenate([row, pad])
    colp = jnp.concatenate([col, pad])

    wr = jnp.concatenate([A_w.T, Dm_w.T], axis=0)
    wc = jnp.concatenate([B_w.T, C_w.T], axis=0)
    br = (A_b + Dm_b).reshape(1, DD)
    bc = (B_b + C_b).reshape(1, DD)
    vb = V_b.reshape(1, DD)
    ub = U_b.reshape(1, DD)

    cw = _kw(rowp, colp)
    xr, xc, hv, ep = _k1(h, e, wr, br, wc, bc, V_w.T, vb)
    st = _k2(rowp, colp, xr, xc)
    gc, gv = _k3a(cw, colp, xc, hv)
    x0 = lax.slice(xr, (0, 0), (1, DD))
    c0 = lax.slice(xc, (0, 0), (1, DD))
    sig, cn = _k3b(xr, gc, gv, ep, st.reshape(NT, 2, DD), x0, c0,
                   bn_g.reshape(1, DD), bn_b.reshape(1, DD))
    h_out, e_new = _k4(h, sig, cn, U_w.T, ub)
    return (h_out, e_new)


# speedup vs baseline: 3.1710x; 3.1710x over previous
import functools

import jax
import jax.numpy as jnp
from jax import lax
from jax.experimental import pallas as pl
from jax.experimental.pallas import tpu as pltpu
from jax.experimental.pallas import tpu_sc as plsc

NN = 100000
EE = 100000
DD = 128
NT = 32
PT = 3136
NP = NT * PT
CH = 448
NCH = PT // CH
CW_CH = 3584
EPAD = NP - EE
BN_EPS = 1e-5
EPS = 1e-5

_f32 = jnp.float32
_mesh = plsc.VectorSubcoreMesh(core_axis_name="c", subcore_axis_name="s")
_sc_params = pltpu.CompilerParams(needs_layout_passes=False)


def _wid():
    return lax.axis_index("s") * 2 + lax.axis_index("c")


def _k1_body(h_ref, e_ref, wr_ref, br_ref, wc_ref, bc_ref, wv_ref, vb_ref,
             xr_ref, xc_ref, hv_ref, ep_ref):
    i = pl.program_id(0)
    rows = i * 512 + lax.broadcasted_iota(jnp.int32, (512, 1), 0)
    m = rows < NN
    h_raw = h_ref[...]
    e_raw = e_ref[...]
    hb = jnp.where(m, h_raw, 0.0)
    eb = jnp.where(m, e_raw, 0.0)
    he = jnp.concatenate([hb, eb], axis=1)
    xr_ref[...] = jnp.dot(he, wr_ref[...], preferred_element_type=_f32) + br_ref[...]
    xc_ref[...] = jnp.where(
        m, jnp.dot(he, wc_ref[...], preferred_element_type=_f32) + bc_ref[...], -1e30)
    hv_ref[...] = jnp.dot(hb, wv_ref[...], preferred_element_type=_f32) + vb_ref[...]
    ep_ref[...] = jnp.where(m, e_raw, -1e30)


def _k1(h, e, wr, br, wc, bc, wv, vb):
    blk = lambda s: pl.BlockSpec(s, lambda i: (0,) * len(s))
    return pl.pallas_call(
        _k1_body,
        grid=(NP // 512,),
        in_specs=[
            pl.BlockSpec((512, DD), lambda i: (i, 0)),
            pl.BlockSpec((512, DD), lambda i: (i, 0)),
            blk((2 * DD, DD)), blk((1, DD)),
            blk((2 * DD, DD)), blk((1, DD)),
            blk((DD, DD)), blk((1, DD)),
        ],
        out_specs=[pl.BlockSpec((512, DD), lambda i: (i, 0))] * 4,
        out_shape=[jax.ShapeDtypeStruct((NP, DD), _f32)] * 4,
    )(h, e, wr, br, wc, bc, wv, vb)


@functools.partial(
    pl.kernel,
    out_type=jax.ShapeDtypeStruct((NP,), jnp.int32),
    mesh=_mesh,
    compiler_params=_sc_params,
    scratch_types=[
        pltpu.VMEM((PT,), jnp.int32),
        pltpu.VMEM((CW_CH,), jnp.int32),
        pltpu.VMEM((CW_CH,), jnp.int32),
    ],
)
def _kw(rowp_hbm, colp_hbm, cw_hbm, slab, rbuf, cbuf):
    wid = _wid()
    base = wid * PT
    neg1 = jnp.full((16,), -1, jnp.int32)
    iota = lax.iota(jnp.int32, 16)

    def init(i, _):
        slab[pl.ds(i * 16, 16)] = neg1
        return 0
    lax.fori_loop(0, PT // 16, init, 0)

    def chunk(cix, _):
        pltpu.sync_copy(rowp_hbm.at[pl.ds(cix * CW_CH, CW_CH)], rbuf)
        pltpu.sync_copy(colp_hbm.at[pl.ds(cix * CW_CH, CW_CH)], cbuf)
        kbase = cix * CW_CH

        def vreg(i, _):
            rv = rbuf[pl.ds(i * 16, 16)]
            cv = cbuf[pl.ds(i * 16, 16)]
            kvec = kbase + i * 16 + iota
            m = (rv >= base) & (rv < base + PT) & (kvec < EE)
            idx = jnp.clip(rv - base, 0, PT - 1)
            plsc.store_scatter(slab, [idx], cv, mask=m)
            return 0
        lax.fori_loop(0, CW_CH // 16, vreg, 0)
        return 0
    lax.fori_loop(0, NP // CW_CH, chunk, 0)
    pltpu.sync_copy(slab, cw_hbm.at[pl.ds(base, PT)])


@functools.partial(
    pl.kernel,
    out_type=jax.ShapeDtypeStruct((NT * 2 * DD,), _f32),
    mesh=_mesh,
    compiler_params=_sc_params,
    scratch_types=[
        pltpu.VMEM((CH,), jnp.int32),
        pltpu.VMEM((CH,), jnp.int32),
        pltpu.VMEM((CH, DD), _f32),
        pltpu.VMEM((CH, DD), _f32),
        pltpu.VMEM((2 * DD,), _f32),
        pltpu.SemaphoreType.DMA,
        pltpu.SemaphoreType.DMA,
    ],
)
def _k2(rowp_hbm, colp_hbm, xr_hbm, xc_hbm, st_hbm,
        ribuf, cibuf, xrb, xcb, outb, sem1, sem2):
    wid = _wid()
    base = wid * PT
    zero = jnp.zeros((16,), _f32)

    def chunk(cix, carry):
        off = base + cix * CH
        pltpu.sync_copy(rowp_hbm.at[pl.ds(off, CH)], ribuf)
        pltpu.sync_copy(colp_hbm.at[pl.ds(off, CH)], cibuf)
        cp1 = pltpu.async_copy(xr_hbm.at[ribuf], xrb, sem1)
        cp2 = pltpu.async_copy(xc_hbm.at[cibuf], xcb, sem2)
        cp1.wait()
        cp2.wait()

        def edge(i, car):
            s = list(car[0])
            q = list(car[1])
            for j in range(8):
                sl = pl.ds(j * 16, 16)
                t = xrb[i, sl] + xcb[i, sl]
                s[j] = s[j] + t
                q[j] = q[j] + t * t
            return (tuple(s), tuple(q))
        return lax.fori_loop(0, CH, edge, carry)

    init = (tuple(zero for _ in range(8)), tuple(zero for _ in range(8)))
    s, q = lax.fori_loop(0, NCH, chunk, init)
    for j in range(8):
        outb[pl.ds(j * 16, 16)] = s[j]
        outb[pl.ds(DD + j * 16, 16)] = q[j]
    pltpu.sync_copy(outb, st_hbm.at[pl.ds(wid * 2 * DD, 2 * DD)])


@functools.partial(
    pl.kernel,
    out_type=(jax.ShapeDtypeStruct((NP, DD), _f32),
              jax.ShapeDtypeStruct((NP, DD), _f32)),
    mesh=_mesh,
    compiler_params=_sc_params,
    scratch_types=[
        pltpu.VMEM((CH,), jnp.int32),
        pltpu.VMEM((CH,), jnp.int32),
        pltpu.VMEM((CH,), jnp.int32),
        pltpu.VMEM((CH, DD), _f32),
        pltpu.VMEM((CH, DD), _f32),
        pltpu.SemaphoreType.DMA,
        pltpu.SemaphoreType.DMA,
    ],
)
def _k3a(cw_hbm, colp_hbm, xc_hbm, hv_hbm, gc_hbm, gv_hbm,
         cwb, colb, idxb, xcgb, gvb, sem1, sem2):
    wid = _wid()
    base = wid * PT
    iota = lax.iota(jnp.int32, 16)

    def chunk(cix, _):
        noff = base + cix * CH
        pltpu.sync_copy(cw_hbm.at[pl.ds(noff, CH)], cwb)
        pltpu.sync_copy(colp_hbm.at[pl.ds(noff, CH)], colb)

        def mk(i, _):
            cwv = cwb[pl.ds(i * 16, 16)]
            spread = NN + ((i * 16 + iota) & 255)
            idxb[pl.ds(i * 16, 16)] = jnp.where(cwv < 0, spread, cwv)
            return 0
        lax.fori_loop(0, CH // 16, mk, 0)
        cp1 = pltpu.async_copy(xc_hbm.at[idxb], xcgb, sem1)
        cp2 = pltpu.async_copy(hv_hbm.at[colb], gvb, sem2)
        cp1.wait()
        cp2.wait()
        pltpu.sync_copy(xcgb, gc_hbm.at[pl.ds(noff, CH)])
        pltpu.sync_copy(gvb, gv_hbm.at[pl.ds(noff, CH)])
        return 0
    lax.fori_loop(0, NCH, chunk, 0)


def _k3b_body(xr_ref, gc_ref, gv_ref, ep_ref, st_ref, x0_ref, c0_ref,
              g_ref, b_ref, sig_ref, cn_ref):
    i = pl.program_id(0)
    stf = st_ref[...]
    ssum = jnp.sum(stf[:, 0, :], axis=0)
    qsum = jnp.sum(stf[:, 1, :], axis=0)
    t0 = (x0_ref[...] + c0_ref[...])[0]
    ssum = ssum - float(EPAD) * t0
    qsum = qsum - float(EPAD) * (t0 * t0)
    mean = ssum * (1.0 / EE)
    var = jnp.maximum(qsum * (1.0 / EE) - mean * mean, 0.0)
    s = g_ref[...][0] * lax.rsqrt(var + BN_EPS)
    c = b_ref[...][0] - mean * s
    t = xr_ref[...] + gc_ref[...]
    add = jnp.maximum(t * s[None, :] + c[None, :], 0.0)
    pre = ep_ref[...] + add
    sig = jax.nn.sigmoid(pre)
    sig_ref[...] = sig
    ps = jnp.sum(sig, axis=0)
    pn = jnp.sum(sig * gv_ref[...], axis=0)
    acc = jnp.stack([ps, pn], axis=0)

    @pl.when(i == 0)
    def _():
        cn_ref[...] = acc

    @pl.when(i > 0)
    def _():
        cn_ref[...] += acc


def _k3b(xr, gc, gv, ep, st, x0, c0, g, b):
    blk = lambda s: pl.BlockSpec(s, lambda i: (0,) * len(s))
    return pl.pallas_call(
        _k3b_body,
        grid=(NP // 512,),
        in_specs=[
            pl.BlockSpec((512, DD), lambda i: (i, 0)),
            pl.BlockSpec((512, DD), lambda i: (i, 0)),
            pl.BlockSpec((512, DD), lambda i: (i, 0)),
            pl.BlockSpec((512, DD), lambda i: (i, 0)),
            blk((NT, 2, DD)), blk((1, DD)), blk((1, DD)),
            blk((1, DD)), blk((1, DD)),
        ],
        out_specs=[pl.BlockSpec((512, DD), lambda i: (i, 0)),
                   pl.BlockSpec((2, DD), lambda i: (0, 0))],
        out_shape=[jax.ShapeDtypeStruct((NP, DD), _f32),
                   jax.ShapeDtypeStruct((2, DD), _f32)],
    )(xr, gc, gv, ep, st, x0, c0, g, b)


def _k4_body(h_ref, sig_ref, cn_ref, wu_ref, ub_ref, hout_ref, enew_ref):
    cn = cn_ref[...]
    r = 1.0 / (cn[0] + EPS)
    enew_ref[...] = sig_ref[...] * r[None, :]
    hu = jnp.dot(h_ref[...], wu_ref[...], preferred_element_type=_f32) + ub_ref[...]
    hout_ref[...] = jnp.maximum(hu + (cn[1] * r)[None, :], 0.0)


def _k4(h, sig, cn, wu, ub):
    return pl.pallas_call(
        _k4_body,
        grid=(NN // 1000,),
        in_specs=[
            pl.BlockSpec((1000, DD), lambda i: (i, 0)),
            pl.BlockSpec((1000, DD), lambda i: (i, 0)),
            pl.BlockSpec((2, DD), lambda i: (0, 0)),
            pl.BlockSpec((DD, DD), lambda i: (0, 0)),
            pl.BlockSpec((1, DD), lambda i: (0, 0)),
        ],
        out_specs=[pl.BlockSpec((1000, DD), lambda i: (i, 0))] * 2,
        out_shape=[jax.ShapeDtypeStruct((NN, DD), _f32)] * 2,
    )(h, sig, cn, wu, ub)


def kernel(h, e, edge_index, A_w, A_b, B_w, B_b, C_w, C_b, Dm_w, Dm_b,
           U_w, U_b, V_w, V_b, bn_g, bn_b):
    row = edge_index[0]
    col = edge_index[1]
    pad = jnp.zeros((NP - EE,), jnp.int32)
    rowp = jnp.concatenate([row, pad])
    colp = jnp.concatenate([col, pad])

    wr = jnp.concatenate([A_w.T, Dm_w.T], axis=0)
    wc = jnp.concatenate([B_w.T, C_w.T], axis=0)
    br = (A_b + Dm_b).reshape(1, DD)
    bc = (B_b + C_b).reshape(1, DD)
    vb = V_b.reshape(1, DD)
    ub = U_b.reshape(1, DD)

    cw = _kw(rowp, colp)
    xr, xc, hv, ep = _k1(h, e, wr, br, wc, bc, V_w.T, vb)
    st = _k2(rowp, colp, xr, xc)
    gc, gv = _k3a(cw, colp, xc, hv)
    x0 = lax.slice(xr, (0, 0), (1, DD))
    c0 = lax.slice(xc, (0, 0), (1, DD))
    sig, cn = _k3b(xr, gc, gv, ep, st.reshape(NT, 2, DD), x0, c0,
                   bn_g.reshape(1, DD), bn_b.reshape(1, DD))
    h_out, e_new = _k4(h, sig, cn, U_w.T, ub)
    return (h_out, e_new)


# speedup vs baseline: 3.2290x; 1.0183x over previous
import functools

import jax
import jax.numpy as jnp
from jax import lax
from jax.experimental import pallas as pl
from jax.experimental.pallas import tpu as pltpu
from jax.experimental.pallas import tpu_sc as plsc

NN = 100000
EE = 100000
DD = 128
NT = 32
PT = 3136
NP = NT * PT
CH = 448
NCH = PT // CH
CW_CH = 7168
EPAD = NP - EE
BN_EPS = 1e-5
EPS = 1e-5

_f32 = jnp.float32
_mesh = plsc.VectorSubcoreMesh(core_axis_name="c", subcore_axis_name="s")
_sc_params = pltpu.CompilerParams(needs_layout_passes=False)


def _wid():
    return lax.axis_index("s") * 2 + lax.axis_index("c")


def _k1_body(h_ref, e_ref, wr_ref, br_ref, wc_ref, bc_ref, wv_ref, vb_ref,
             xr_ref, xc_ref, hv_ref):
    i = pl.program_id(0)
    rows = i * 512 + lax.broadcasted_iota(jnp.int32, (512, 1), 0)
    m = rows < NN
    hb = jnp.where(m, h_ref[...], 0.0)
    eb = jnp.where(m, e_ref[...], 0.0)
    he = jnp.concatenate([hb, eb], axis=1)
    xr_ref[...] = jnp.dot(he, wr_ref[...], preferred_element_type=_f32) + br_ref[...]
    xc_ref[...] = jnp.where(
        m, jnp.dot(he, wc_ref[...], preferred_element_type=_f32) + bc_ref[...], -1e30)
    hv_ref[...] = jnp.dot(hb, wv_ref[...], preferred_element_type=_f32) + vb_ref[...]


def _k1(h, e, wr, br, wc, bc, wv, vb):
    blk = lambda s: pl.BlockSpec(s, lambda i: (0,) * len(s))
    return pl.pallas_call(
        _k1_body,
        grid=(NP // 512,),
        in_specs=[
            pl.BlockSpec((512, DD), lambda i: (i, 0)),
            pl.BlockSpec((512, DD), lambda i: (i, 0)),
            blk((2 * DD, DD)), blk((1, DD)),
            blk((2 * DD, DD)), blk((1, DD)),
            blk((DD, DD)), blk((1, DD)),
        ],
        out_specs=[pl.BlockSpec((512, DD), lambda i: (i, 0))] * 3,
        out_shape=[jax.ShapeDtypeStruct((NP, DD), _f32)] * 3,
    )(h, e, wr, br, wc, bc, wv, vb)


@functools.partial(
    pl.kernel,
    out_type=jax.ShapeDtypeStruct((NP,), jnp.int32),
    mesh=_mesh,
    compiler_params=_sc_params,
    scratch_types=[
        pltpu.VMEM((PT,), jnp.int32),
        pltpu.VMEM((CW_CH,), jnp.int32),
        pltpu.VMEM((CW_CH,), jnp.int32),
    ],
)
def _kw(rowp_hbm, colp_hbm, cw_hbm, slab, rbuf, cbuf):
    wid = _wid()
    base = wid * PT
    neg1 = jnp.full((16,), -1, jnp.int32)
    iota = lax.iota(jnp.int32, 16)

    def init(i, _):
        slab[pl.ds(i * 16, 16)] = neg1
        return 0
    lax.fori_loop(0, PT // 16, init, 0)

    def chunk(cix, _):
        pltpu.sync_copy(rowp_hbm.at[pl.ds(cix * CW_CH, CW_CH)], rbuf)
        pltpu.sync_copy(colp_hbm.at[pl.ds(cix * CW_CH, CW_CH)], cbuf)
        kbase = cix * CW_CH

        def vreg(i, _):
            rv = rbuf[pl.ds(i * 16, 16)]
            cv = cbuf[pl.ds(i * 16, 16)]
            kvec = kbase + i * 16 + iota
            m = (rv >= base) & (rv < base + PT) & (kvec < EE)
            idx = jnp.clip(rv - base, 0, PT - 1)
            plsc.store_scatter(slab, [idx], cv, mask=m)
            return 0
        lax.fori_loop(0, CW_CH // 16, vreg, 0)
        return 0
    lax.fori_loop(0, NP // CW_CH, chunk, 0)
    pltpu.sync_copy(slab, cw_hbm.at[pl.ds(base, PT)])


@functools.partial(
    pl.kernel,
    out_type=(jax.ShapeDtypeStruct((NT * 2 * DD,), _f32),
              jax.ShapeDtypeStruct((NP, DD), _f32),
              jax.ShapeDtypeStruct((NP, DD), _f32)),
    mesh=_mesh,
    compiler_params=_sc_params,
    scratch_types=[
        pltpu.VMEM((CH,), jnp.int32),
        pltpu.VMEM((CH,), jnp.int32),
        pltpu.VMEM((CH,), jnp.int32),
        pltpu.VMEM((CH, DD), _f32),
        pltpu.VMEM((CH, DD), _f32),
        pltpu.VMEM((2 * DD,), _f32),
        pltpu.SemaphoreType.DMA,
        pltpu.SemaphoreType.DMA,
    ],
)
def _k23(rowp_hbm, colp_hbm, cw_hbm, xr_hbm, xc_hbm, hv_hbm,
         st_hbm, gc_hbm, gv_hbm,
         ribuf, cibuf, idxb, bufa, bufb, outb, sem1, sem2):
    wid = _wid()
    base = wid * PT
    zero = jnp.zeros((16,), _f32)
    iota = lax.iota(jnp.int32, 16)

    def chunk_a(cix, carry):
        off = base + cix * CH
        pltpu.sync_copy(rowp_hbm.at[pl.ds(off, CH)], ribuf)
        pltpu.sync_copy(colp_hbm.at[pl.ds(off, CH)], cibuf)
        cp1 = pltpu.async_copy(xr_hbm.at[ribuf], bufa, sem1)
        cp2 = pltpu.async_copy(xc_hbm.at[cibuf], bufb, sem2)
        cp1.wait()
        cp2.wait()

        def edge(i, car):
            s = list(car[0])
            q = list(car[1])
            for j in range(8):
                sl = pl.ds(j * 16, 16)
                t = bufa[i, sl] + bufb[i, sl]
                s[j] = s[j] + t
                q[j] = q[j] + t * t
            return (tuple(s), tuple(q))
        return lax.fori_loop(0, CH, edge, carry)

    init = (tuple(zero for _ in range(8)), tuple(zero for _ in range(8)))
    s, q = lax.fori_loop(0, NCH, chunk_a, init)
    for j in range(8):
        outb[pl.ds(j * 16, 16)] = s[j]
        outb[pl.ds(DD + j * 16, 16)] = q[j]
    pltpu.sync_copy(outb, st_hbm.at[pl.ds(wid * 2 * DD, 2 * DD)])

    def chunk_b(cix, _):
        noff = base + cix * CH
        pltpu.sync_copy(cw_hbm.at[pl.ds(noff, CH)], ribuf)
        pltpu.sync_copy(colp_hbm.at[pl.ds(noff, CH)], cibuf)

        def mk(i, _):
            cwv = ribuf[pl.ds(i * 16, 16)]
            spread = NN + ((i * 16 + iota) & 255)
            idxb[pl.ds(i * 16, 16)] = jnp.where(cwv < 0, spread, cwv)
            return 0
        lax.fori_loop(0, CH // 16, mk, 0)
        cp1 = pltpu.async_copy(xc_hbm.at[idxb], bufa, sem1)
        cp2 = pltpu.async_copy(hv_hbm.at[cibuf], bufb, sem2)
        cp1.wait()
        cp2.wait()
        pltpu.sync_copy(bufa, gc_hbm.at[pl.ds(noff, CH)])
        pltpu.sync_copy(bufb, gv_hbm.at[pl.ds(noff, CH)])
        return 0
    lax.fori_loop(0, NCH, chunk_b, 0)


def _k3b_body(xr_ref, gc_ref, gv_ref, e_ref, st_ref, x0_ref, c0_ref,
              g_ref, b_ref, sig_ref, cn_ref):
    i = pl.program_id(0)
    rows = i * 512 + lax.broadcasted_iota(jnp.int32, (512, 1), 0)
    stf = st_ref[...]
    ssum = jnp.sum(stf[:, 0, :], axis=0)
    qsum = jnp.sum(stf[:, 1, :], axis=0)
    t0 = (x0_ref[...] + c0_ref[...])[0]
    ssum = ssum - float(EPAD) * t0
    qsum = qsum - float(EPAD) * (t0 * t0)
    mean = ssum * (1.0 / EE)
    var = jnp.maximum(qsum * (1.0 / EE) - mean * mean, 0.0)
    s = g_ref[...][0] * lax.rsqrt(var + BN_EPS)
    c = b_ref[...][0] - mean * s
    t = xr_ref[...] + gc_ref[...]
    add = jnp.maximum(t * s[None, :] + c[None, :], 0.0)
    pre = jnp.where(rows < NN, e_ref[...] + add, -1e30)
    sig = jax.nn.sigmoid(pre)
    sig_ref[...] = sig
    ps = jnp.sum(sig, axis=0)
    pn = jnp.sum(sig * gv_ref[...], axis=0)
    acc = jnp.stack([ps, pn], axis=0)

    @pl.when(i == 0)
    def _():
        cn_ref[...] = acc

    @pl.when(i > 0)
    def _():
        cn_ref[...] += acc


def _k3b(xr, gc, gv, e, st, x0, c0, g, b):
    blk = lambda s: pl.BlockSpec(s, lambda i: (0,) * len(s))
    return pl.pallas_call(
        _k3b_body,
        grid=(NP // 512,),
        in_specs=[
            pl.BlockSpec((512, DD), lambda i: (i, 0)),
            pl.BlockSpec((512, DD), lambda i: (i, 0)),
            pl.BlockSpec((512, DD), lambda i: (i, 0)),
            pl.BlockSpec((512, DD), lambda i: (i, 0)),
            blk((NT, 2, DD)), blk((1, DD)), blk((1, DD)),
            blk((1, DD)), blk((1, DD)),
        ],
        out_specs=[pl.BlockSpec((512, DD), lambda i: (i, 0)),
                   pl.BlockSpec((2, DD), lambda i: (0, 0))],
        out_shape=[jax.ShapeDtypeStruct((NP, DD), _f32),
                   jax.ShapeDtypeStruct((2, DD), _f32)],
    )(xr, gc, gv, e, st, x0, c0, g, b)


def _k4_body(h_ref, sig_ref, cn_ref, wu_ref, ub_ref, hout_ref, enew_ref):
    cn = cn_ref[...]
    r = 1.0 / (cn[0] + EPS)
    enew_ref[...] = sig_ref[...] * r[None, :]
    hu = jnp.dot(h_ref[...], wu_ref[...], preferred_element_type=_f32) + ub_ref[...]
    hout_ref[...] = jnp.maximum(hu + (cn[1] * r)[None, :], 0.0)


def _k4(h, sig, cn, wu, ub):
    return pl.pallas_call(
        _k4_body,
        grid=(NN // 1000,),
        in_specs=[
            pl.BlockSpec((1000, DD), lambda i: (i, 0)),
            pl.BlockSpec((1000, DD), lambda i: (i, 0)),
            pl.BlockSpec((2, DD), lambda i: (0, 0)),
            pl.BlockSpec((DD, DD), lambda i: (0, 0)),
            pl.BlockSpec((1, DD), lambda i: (0, 0)),
        ],
        out_specs=[pl.BlockSpec((1000, DD), lambda i: (i, 0))] * 2,
        out_shape=[jax.ShapeDtypeStruct((NN, DD), _f32)] * 2,
    )(h, sig, cn, wu, ub)


def kernel(h, e, edge_index, A_w, A_b, B_w, B_b, C_w, C_b, Dm_w, Dm_b,
           U_w, U_b, V_w, V_b, bn_g, bn_b):
    row = edge_index[0]
    col = edge_index[1]
    pad = jnp.zeros((NP - EE,), jnp.int32)
    rowp = jnp.concatenate([row, pad])
    colp = jnp.concatenate([col, pad])

    wr = jnp.concatenate([A_w.T, Dm_w.T], axis=0)
    wc = jnp.concatenate([B_w.T, C_w.T], axis=0)
    br = (A_b + Dm_b).reshape(1, DD)
    bc = (B_b + C_b).reshape(1, DD)
    vb = V_b.reshape(1, DD)
    ub = U_b.reshape(1, DD)

    cw = _kw(rowp, colp)
    xr, xc, hv = _k1(h, e, wr, br, wc, bc, V_w.T, vb)
    st, gc, gv = _k23(rowp, colp, cw, xr, xc, hv)
    x0 = lax.slice(xr, (0, 0), (1, DD))
    c0 = lax.slice(xc, (0, 0), (1, DD))
    sig, cn = _k3b(xr, gc, gv, e, st.reshape(NT, 2, DD), x0, c0,
                   bn_g.reshape(1, DD), bn_b.reshape(1, DD))
    h_out, e_new = _k4(h, sig, cn, U_w.T, ub)
    return (h_out, e_new)


# speedup vs baseline: 4.0178x; 1.2443x over previous
import functools

import jax
import jax.numpy as jnp
from jax import lax
from jax.experimental import pallas as pl
from jax.experimental.pallas import tpu as pltpu
from jax.experimental.pallas import tpu_sc as plsc

NN = 100000
EE = 100000
DD = 128
NT = 32
PT = 3136
NP = NT * PT
CH = 448
NCH = PT // CH
CW_CH = 7168
EPAD = NP - EE
BN_EPS = 1e-5
EPS = 1e-5

_f32 = jnp.float32
_mesh = plsc.VectorSubcoreMesh(core_axis_name="c", subcore_axis_name="s")
_sc_params = pltpu.CompilerParams(needs_layout_passes=False)
_DN_T = (((1,), (1,)), ((), ()))


def _wid():
    return lax.axis_index("s") * 2 + lax.axis_index("c")


@functools.partial(
    pl.kernel,
    out_type=(jax.ShapeDtypeStruct((NP,), jnp.int32),
              jax.ShapeDtypeStruct((NP, DD), _f32)),
    mesh=_mesh,
    compiler_params=_sc_params,
    scratch_types=[
        pltpu.VMEM((PT,), jnp.int32),
        pltpu.VMEM((CW_CH,), jnp.int32),
        pltpu.VMEM((CW_CH,), jnp.int32),
        pltpu.VMEM((CH,), jnp.int32),
        pltpu.VMEM((CH, DD), _f32),
        pltpu.SemaphoreType.DMA,
    ],
)
def _k0(rowp_hbm, colp_hbm, h_hbm, cw_hbm, gh_hbm,
        slab, rbuf, cbuf, cibuf, bufa, sem1):
    wid = _wid()
    base = wid * PT
    neg1 = jnp.full((16,), -1, jnp.int32)
    iota = lax.iota(jnp.int32, 16)

    def chunk_g(cix, _):
        noff = base + cix * CH
        pltpu.sync_copy(colp_hbm.at[pl.ds(noff, CH)], cibuf)
        pltpu.async_copy(h_hbm.at[cibuf], bufa, sem1).wait()
        pltpu.sync_copy(bufa, gh_hbm.at[pl.ds(noff, CH)])
        return 0
    lax.fori_loop(0, NCH, chunk_g, 0)

    def init(i, _):
        slab[pl.ds(i * 16, 16)] = neg1
        return 0
    lax.fori_loop(0, PT // 16, init, 0)

    def chunk_w(cix, _):
        pltpu.sync_copy(rowp_hbm.at[pl.ds(cix * CW_CH, CW_CH)], rbuf)
        pltpu.sync_copy(colp_hbm.at[pl.ds(cix * CW_CH, CW_CH)], cbuf)
        kbase = cix * CW_CH

        def vreg(i, _):
            rv = rbuf[pl.ds(i * 16, 16)]
            cv = cbuf[pl.ds(i * 16, 16)]
            kvec = kbase + i * 16 + iota
            m = (rv >= base) & (rv < base + PT) & (kvec < EE)
            idx = jnp.clip(rv - base, 0, PT - 1)
            plsc.store_scatter(slab, [idx], cv, mask=m)
            return 0
        lax.fori_loop(0, CW_CH // 16, vreg, 0)
        return 0
    lax.fori_loop(0, NP // CW_CH, chunk_w, 0)
    pltpu.sync_copy(slab, cw_hbm.at[pl.ds(base, PT)])


def _k1_body(h_ref, e_ref, wa_ref, wd_ref, wb_ref, wcw_ref, br_ref, bc_ref,
             xr_ref, xc_ref):
    i = pl.program_id(0)
    rows = i * 1024 + lax.broadcasted_iota(jnp.int32, (1024, 1), 0)
    m = rows < NN
    hb = h_ref[...]
    eb = e_ref[...]
    xr_ref[...] = (lax.dot_general(hb, wa_ref[...], _DN_T, preferred_element_type=_f32)
                   + lax.dot_general(eb, wd_ref[...], _DN_T, preferred_element_type=_f32)
                   + br_ref[...])
    xc = (lax.dot_general(hb, wb_ref[...], _DN_T, preferred_element_type=_f32)
          + lax.dot_general(eb, wcw_ref[...], _DN_T, preferred_element_type=_f32)
          + bc_ref[...])
    xc_ref[...] = jnp.where(m, xc, -1e30)


def _k1(h, e, wa, wd, wb, wcw, br, bc):
    blk = lambda s: pl.BlockSpec(s, lambda i: (0,) * len(s))
    return pl.pallas_call(
        _k1_body,
        grid=(NP // 1024,),
        in_specs=[
            pl.BlockSpec((1024, DD), lambda i: (i, 0)),
            pl.BlockSpec((1024, DD), lambda i: (i, 0)),
            blk((DD, DD)), blk((DD, DD)), blk((DD, DD)), blk((DD, DD)),
            blk((1, DD)), blk((1, DD)),
        ],
        out_specs=[pl.BlockSpec((1024, DD), lambda i: (i, 0))] * 2,
        out_shape=[jax.ShapeDtypeStruct((NP, DD), _f32)] * 2,
    )(h, e, wa, wd, wb, wcw, br, bc)


@functools.partial(
    pl.kernel,
    out_type=(jax.ShapeDtypeStruct((NT * 2 * DD,), _f32),
              jax.ShapeDtypeStruct((NP, DD), _f32)),
    mesh=_mesh,
    compiler_params=_sc_params,
    scratch_types=[
        pltpu.VMEM((CH,), jnp.int32),
        pltpu.VMEM((CH,), jnp.int32),
        pltpu.VMEM((CH,), jnp.int32),
        pltpu.VMEM((CH, DD), _f32),
        pltpu.VMEM((CH, DD), _f32),
        pltpu.VMEM((2 * DD,), _f32),
        pltpu.SemaphoreType.DMA,
        pltpu.SemaphoreType.DMA,
    ],
)
def _k2(rowp_hbm, colp_hbm, cw_hbm, xr_hbm, xc_hbm,
        st_hbm, gc_hbm,
        ribuf, cibuf, idxb, bufa, bufb, outb, sem1, sem2):
    wid = _wid()
    base = wid * PT
    zero = jnp.zeros((16,), _f32)
    iota = lax.iota(jnp.int32, 16)

    def chunk_a(cix, carry):
        off = base + cix * CH
        pltpu.sync_copy(rowp_hbm.at[pl.ds(off, CH)], ribuf)
        pltpu.sync_copy(colp_hbm.at[pl.ds(off, CH)], cibuf)
        cp1 = pltpu.async_copy(xr_hbm.at[ribuf], bufa, sem1)
        cp2 = pltpu.async_copy(xc_hbm.at[cibuf], bufb, sem2)
        cp1.wait()
        cp2.wait()

        def edge(i, car):
            s = list(car[0])
            q = list(car[1])
            for j in range(8):
                sl = pl.ds(j * 16, 16)
                t = bufa[i, sl] + bufb[i, sl]
                s[j] = s[j] + t
                q[j] = q[j] + t * t
            return (tuple(s), tuple(q))
        return lax.fori_loop(0, CH, edge, carry)

    init = (tuple(zero for _ in range(8)), tuple(zero for _ in range(8)))
    s, q = lax.fori_loop(0, NCH, chunk_a, init)
    for j in range(8):
        outb[pl.ds(j * 16, 16)] = s[j]
        outb[pl.ds(DD + j * 16, 16)] = q[j]
    pltpu.sync_copy(outb, st_hbm.at[pl.ds(wid * 2 * DD, 2 * DD)])

    def chunk_b(cix, _):
        noff = base + cix * CH
        pltpu.sync_copy(cw_hbm.at[pl.ds(noff, CH)], ribuf)

        def mk(i, _):
            cwv = ribuf[pl.ds(i * 16, 16)]
            spread = NN + ((i * 16 + iota) & 255)
            idxb[pl.ds(i * 16, 16)] = jnp.where(cwv < 0, spread, cwv)
            return 0
        lax.fori_loop(0, CH // 16, mk, 0)
        pltpu.async_copy(xc_hbm.at[idxb], bufa, sem1).wait()
        pltpu.sync_copy(bufa, gc_hbm.at[pl.ds(noff, CH)])
        return 0
    lax.fori_loop(0, NCH, chunk_b, 0)


def _k3_body(xr_ref, gc_ref, gh_ref, e_ref, st_ref, x0_ref, c0_ref,
             g_ref, b_ref, wv_ref, vb_ref, sig_ref, cn_ref):
    i = pl.program_id(0)
    rows = i * 1024 + lax.broadcasted_iota(jnp.int32, (1024, 1), 0)
    stf = st_ref[...]
    ssum = jnp.sum(stf[:, 0, :], axis=0)
    qsum = jnp.sum(stf[:, 1, :], axis=0)
    t0 = (x0_ref[...] + c0_ref[...])[0]
    ssum = ssum - float(EPAD) * t0
    qsum = qsum - float(EPAD) * (t0 * t0)
    mean = ssum * (1.0 / EE)
    var = jnp.maximum(qsum * (1.0 / EE) - mean * mean, 0.0)
    s = g_ref[...][0] * lax.rsqrt(var + BN_EPS)
    c = b_ref[...][0] - mean * s
    t = xr_ref[...] + gc_ref[...]
    add = jnp.maximum(t * s[None, :] + c[None, :], 0.0)
    pre = jnp.where(rows < NN, e_ref[...] + add, -1e30)
    sig = jax.nn.sigmoid(pre)
    sig_ref[...] = sig
    gv = (lax.dot_general(gh_ref[...], wv_ref[...], _DN_T,
                          preferred_element_type=_f32) + vb_ref[...])
    ps = jnp.sum(sig, axis=0)
    pn = jnp.sum(sig * gv, axis=0)
    acc = jnp.stack([ps, pn], axis=0)

    @pl.when(i == 0)
    def _():
        cn_ref[...] = acc

    @pl.when(i > 0)
    def _():
        cn_ref[...] += acc


def _k3(xr, gc, gh, e, st, x0, c0, g, b, wv, vb):
    blk = lambda s: pl.BlockSpec(s, lambda i: (0,) * len(s))
    return pl.pallas_call(
        _k3_body,
        grid=(NP // 1024,),
        in_specs=[
            pl.BlockSpec((1024, DD), lambda i: (i, 0)),
            pl.BlockSpec((1024, DD), lambda i: (i, 0)),
            pl.BlockSpec((1024, DD), lambda i: (i, 0)),
            pl.BlockSpec((1024, DD), lambda i: (i, 0)),
            blk((NT, 2, DD)), blk((1, DD)), blk((1, DD)),
            blk((1, DD)), blk((1, DD)), blk((DD, DD)), blk((1, DD)),
        ],
        out_specs=[pl.BlockSpec((1024, DD), lambda i: (i, 0)),
                   pl.BlockSpec((2, DD), lambda i: (0, 0))],
        out_shape=[jax.ShapeDtypeStruct((NP, DD), _f32),
                   jax.ShapeDtypeStruct((2, DD), _f32)],
    )(xr, gc, gh, e, st, x0, c0, g, b, wv, vb)


def _k4_body(h_ref, sig_ref, cn_ref, wu_ref, ub_ref, hout_ref, enew_ref):
    cn = cn_ref[...]
    r = 1.0 / (cn[0] + EPS)
    enew_ref[...] = sig_ref[...] * r[None, :]
    hu = (lax.dot_general(h_ref[...], wu_ref[...], _DN_T,
                          preferred_element_type=_f32) + ub_ref[...])
    hout_ref[...] = jnp.maximum(hu + (cn[1] * r)[None, :], 0.0)


def _k4(h, sig, cn, wu, ub):
    return pl.pallas_call(
        _k4_body,
        grid=(NN // 2000,),
        in_specs=[
            pl.BlockSpec((2000, DD), lambda i: (i, 0)),
            pl.BlockSpec((2000, DD), lambda i: (i, 0)),
            pl.BlockSpec((2, DD), lambda i: (0, 0)),
            pl.BlockSpec((DD, DD), lambda i: (0, 0)),
            pl.BlockSpec((1, DD), lambda i: (0, 0)),
        ],
        out_specs=[pl.BlockSpec((2000, DD), lambda i: (i, 0))] * 2,
        out_shape=[jax.ShapeDtypeStruct((NN, DD), _f32)] * 2,
    )(h, sig, cn, wu, ub)


def kernel(h, e, edge_index, A_w, A_b, B_w, B_b, C_w, C_b, Dm_w, Dm_b,
           U_w, U_b, V_w, V_b, bn_g, bn_b):
    row = edge_index[0]
    col = edge_index[1]
    pad = jnp.zeros((NP - EE,), jnp.int32)
    rowp = jnp.concatenate([row, pad])
    colp = jnp.concatenate([col, pad])

    br = (A_b + Dm_b).reshape(1, DD)
    bc = (B_b + C_b).reshape(1, DD)
    vb = V_b.reshape(1, DD)
    ub = U_b.reshape(1, DD)

    cw, gh = _k0(rowp, colp, h)
    xr, xc = _k1(h, e, A_w, Dm_w, B_w, C_w, br, bc)
    st, gc = _k2(rowp, colp, cw, xr, xc)
    x0 = lax.slice(xr, (0, 0), (1, DD))
    c0 = lax.slice(xc, (0, 0), (1, DD))
    sig, cn = _k3(xr, gc, gh, e, st.reshape(NT, 2, DD), x0, c0,
                  bn_g.reshape(1, DD), bn_b.reshape(1, DD), V_w, vb)
    h_out, e_new = _k4(h, sig, cn, U_w, ub)
    return (h_out, e_new)


# speedup vs baseline: 4.3505x; 1.0828x over previous
import functools

import jax
import jax.numpy as jnp
from jax import lax
from jax.experimental import pallas as pl
from jax.experimental.pallas import tpu as pltpu
from jax.experimental.pallas import tpu_sc as plsc

NN = 100000
EE = 100000
DD = 128
NT = 32
PT = 3136
NP = NT * PT
CH = 224
NCH = PT // CH
CW_CH = 14336
EPAD = NP - EE
BN_EPS = 1e-5
EPS = 1e-5

_f32 = jnp.float32
_mesh = plsc.VectorSubcoreMesh(core_axis_name="c", subcore_axis_name="s")
_sc_params = pltpu.CompilerParams(needs_layout_passes=False)
_DN_T = (((1,), (1,)), ((), ()))


def _wid():
    return lax.axis_index("s") * 2 + lax.axis_index("c")


@functools.partial(
    pl.kernel,
    out_type=(jax.ShapeDtypeStruct((NP,), jnp.int32),
              jax.ShapeDtypeStruct((NP, DD), _f32)),
    mesh=_mesh,
    compiler_params=_sc_params,
    scratch_types=[
        pltpu.VMEM((PT,), jnp.int32),
        pltpu.VMEM((CW_CH,), jnp.int32),
        pltpu.VMEM((CW_CH,), jnp.int32),
        pltpu.VMEM((CH,), jnp.int32),
        pltpu.VMEM((CH,), jnp.int32),
        pltpu.VMEM((CH, DD), _f32),
        pltpu.VMEM((CH, DD), _f32),
        pltpu.SemaphoreType.DMA,
        pltpu.SemaphoreType.DMA,
    ],
)
def _k0(rowp_hbm, colp_hbm, h_hbm, cw_hbm, gh_hbm,
        slab, rbuf, cbuf, ci0, ci1, ba0, ba1, sem0, sem1):
    wid = _wid()
    base = wid * PT
    neg1 = jnp.full((16,), -1, jnp.int32)
    iota = lax.iota(jnp.int32, 16)

    sets = [(ci0, ba0, sem0), (ci1, ba1, sem1)]

    def issue_g(cix, S):
        ci, ba, sm = S
        noff = base + cix * CH
        pltpu.sync_copy(colp_hbm.at[pl.ds(noff, CH)], ci)
        pltpu.async_copy(h_hbm.at[ci], ba, sm)

    issue_g(0, sets[0])
    for cix in range(NCH):
        ci, ba, sm = sets[cix & 1]
        if cix + 1 < NCH:
            issue_g(cix + 1, sets[(cix + 1) & 1])
        pltpu.make_async_copy(h_hbm.at[ci], ba, sm).wait()
        pltpu.sync_copy(ba, gh_hbm.at[pl.ds(base + cix * CH, CH)])

    def init(i, _):
        slab[pl.ds(i * 16, 16)] = neg1
        return 0
    lax.fori_loop(0, PT // 16, init, 0)

    def chunk_w(cix, _):
        pltpu.sync_copy(rowp_hbm.at[pl.ds(cix * CW_CH, CW_CH)], rbuf)
        pltpu.sync_copy(colp_hbm.at[pl.ds(cix * CW_CH, CW_CH)], cbuf)
        kbase = cix * CW_CH

        def vreg(i, _):
            rv = rbuf[pl.ds(i * 16, 16)]
            cv = cbuf[pl.ds(i * 16, 16)]
            kvec = kbase + i * 16 + iota
            m = (rv >= base) & (rv < base + PT) & (kvec < EE)
            idx = jnp.clip(rv - base, 0, PT - 1)
            plsc.store_scatter(slab, [idx], cv, mask=m)
            return 0
        lax.fori_loop(0, CW_CH // 16, vreg, 0)
        return 0
    lax.fori_loop(0, NP // CW_CH, chunk_w, 0)
    pltpu.sync_copy(slab, cw_hbm.at[pl.ds(base, PT)])


def _k1_body(h_ref, e_ref, wa_ref, wd_ref, wb_ref, wcw_ref, br_ref, bc_ref,
             xr_ref, xc_ref):
    i = pl.program_id(0)
    rows = i * 1024 + lax.broadcasted_iota(jnp.int32, (1024, 1), 0)
    m = rows < NN
    hb = h_ref[...]
    eb = e_ref[...]
    xr_ref[...] = (lax.dot_general(hb, wa_ref[...], _DN_T, preferred_element_type=_f32)
                   + lax.dot_general(eb, wd_ref[...], _DN_T, preferred_element_type=_f32)
                   + br_ref[...])
    xc = (lax.dot_general(hb, wb_ref[...], _DN_T, preferred_element_type=_f32)
          + lax.dot_general(eb, wcw_ref[...], _DN_T, preferred_element_type=_f32)
          + bc_ref[...])
    xc_ref[...] = jnp.where(m, xc, -1e30)


def _k1(h, e, wa, wd, wb, wcw, br, bc):
    blk = lambda s: pl.BlockSpec(s, lambda i: (0,) * len(s))
    return pl.pallas_call(
        _k1_body,
        grid=(NP // 1024,),
        in_specs=[
            pl.BlockSpec((1024, DD), lambda i: (i, 0)),
            pl.BlockSpec((1024, DD), lambda i: (i, 0)),
            blk((DD, DD)), blk((DD, DD)), blk((DD, DD)), blk((DD, DD)),
            blk((1, DD)), blk((1, DD)),
        ],
        out_specs=[pl.BlockSpec((1024, DD), lambda i: (i, 0))] * 2,
        out_shape=[jax.ShapeDtypeStruct((NP, DD), _f32)] * 2,
    )(h, e, wa, wd, wb, wcw, br, bc)


@functools.partial(
    pl.kernel,
    out_type=(jax.ShapeDtypeStruct((NT * 2 * DD,), _f32),
              jax.ShapeDtypeStruct((NP, DD), _f32)),
    mesh=_mesh,
    compiler_params=_sc_params,
    scratch_types=[
        pltpu.VMEM((CH,), jnp.int32),
        pltpu.VMEM((CH,), jnp.int32),
        pltpu.VMEM((CH,), jnp.int32),
        pltpu.VMEM((CH,), jnp.int32),
        pltpu.VMEM((CH, DD), _f32),
        pltpu.VMEM((CH, DD), _f32),
        pltpu.VMEM((CH, DD), _f32),
        pltpu.VMEM((CH, DD), _f32),
        pltpu.VMEM((2 * DD,), _f32),
        pltpu.SemaphoreType.DMA,
        pltpu.SemaphoreType.DMA,
        pltpu.SemaphoreType.DMA,
        pltpu.SemaphoreType.DMA,
    ],
)
def _k2(rowp_hbm, colp_hbm, cw_hbm, xr_hbm, xc_hbm,
        st_hbm, gc_hbm,
        ri0, ci0, ri1, ci1, ba0, bb0, ba1, bb1, outb,
        sa0, sb0, sa1, sb1):
    wid = _wid()
    base = wid * PT
    zero = jnp.zeros((16,), _f32)
    iota = lax.iota(jnp.int32, 16)

    sets = [(ri0, ci0, ba0, bb0, sa0, sb0), (ri1, ci1, ba1, bb1, sa1, sb1)]

    def issue_a(cix, S):
        ri, ci, ba, bb, sa, sb = S
        off = base + cix * CH
        pltpu.sync_copy(rowp_hbm.at[pl.ds(off, CH)], ri)
        pltpu.sync_copy(colp_hbm.at[pl.ds(off, CH)], ci)
        pltpu.async_copy(xr_hbm.at[ri], ba, sa)
        pltpu.async_copy(xc_hbm.at[ci], bb, sb)

    issue_a(0, sets[0])
    carry = (tuple(zero for _ in range(8)), tuple(zero for _ in range(8)))
    for cix in range(NCH):
        ri, ci, ba, bb, sa, sb = sets[cix & 1]
        if cix + 1 < NCH:
            issue_a(cix + 1, sets[(cix + 1) & 1])
        pltpu.make_async_copy(xr_hbm.at[ri], ba, sa).wait()
        pltpu.make_async_copy(xc_hbm.at[ci], bb, sb).wait()

        def edge(i, car):
            s = list(car[0])
            q = list(car[1])
            for j in range(8):
                sl = pl.ds(j * 16, 16)
                t = ba[i, sl] + bb[i, sl]
                s[j] = s[j] + t
                q[j] = q[j] + t * t
            return (tuple(s), tuple(q))
        carry = lax.fori_loop(0, CH, edge, carry)
    s, q = carry
    for j in range(8):
        outb[pl.ds(j * 16, 16)] = s[j]
        outb[pl.ds(DD + j * 16, 16)] = q[j]
    pltpu.sync_copy(outb, st_hbm.at[pl.ds(wid * 2 * DD, 2 * DD)])

    def issue_b(cix, S):
        ri, ci, ba, bb, sa, sb = S
        noff = base + cix * CH
        pltpu.sync_copy(cw_hbm.at[pl.ds(noff, CH)], ri)

        def mk(i, _):
            cwv = ri[pl.ds(i * 16, 16)]
            spread = NN + ((i * 16 + iota) & 255)
            ci[pl.ds(i * 16, 16)] = jnp.where(cwv < 0, spread, cwv)
            return 0
        lax.fori_loop(0, CH // 16, mk, 0)
        pltpu.async_copy(xc_hbm.at[ci], ba, sa)

    issue_b(0, sets[0])
    for cix in range(NCH):
        ri, ci, ba, bb, sa, sb = sets[cix & 1]
        if cix + 1 < NCH:
            issue_b(cix + 1, sets[(cix + 1) & 1])
        pltpu.make_async_copy(xc_hbm.at[ci], ba, sa).wait()
        pltpu.sync_copy(ba, gc_hbm.at[pl.ds(base + cix * CH, CH)])


def _k3_body(xr_ref, gc_ref, gh_ref, e_ref, st_ref, x0_ref, c0_ref,
             g_ref, b_ref, wv_ref, vb_ref, sig_ref, cn_ref):
    i = pl.program_id(0)
    rows = i * 1024 + lax.broadcasted_iota(jnp.int32, (1024, 1), 0)
    stf = st_ref[...]
    ssum = jnp.sum(stf[:, 0, :], axis=0)
    qsum = jnp.sum(stf[:, 1, :], axis=0)
    t0 = (x0_ref[...] + c0_ref[...])[0]
    ssum = ssum - float(EPAD) * t0
    qsum = qsum - float(EPAD) * (t0 * t0)
    mean = ssum * (1.0 / EE)
    var = jnp.maximum(qsum * (1.0 / EE) - mean * mean, 0.0)
    s = g_ref[...][0] * lax.rsqrt(var + BN_EPS)
    c = b_ref[...][0] - mean * s
    t = xr_ref[...] + gc_ref[...]
    add = jnp.maximum(t * s[None, :] + c[None, :], 0.0)
    pre = jnp.where(rows < NN, e_ref[...] + add, -1e30)
    sig = jax.nn.sigmoid(pre)
    sig_ref[...] = sig
    gv = (lax.dot_general(gh_ref[...], wv_ref[...], _DN_T,
                          preferred_element_type=_f32) + vb_ref[...])
    ps = jnp.sum(sig, axis=0)
    pn = jnp.sum(sig * gv, axis=0)
    acc = jnp.stack([ps, pn], axis=0)

    @pl.when(i == 0)
    def _():
        cn_ref[...] = acc

    @pl.when(i > 0)
    def _():
        cn_ref[...] += acc


def _k3(xr, gc, gh, e, st, x0, c0, g, b, wv, vb):
    blk = lambda s: pl.BlockSpec(s, lambda i: (0,) * len(s))
    return pl.pallas_call(
        _k3_body,
        grid=(NP // 1024,),
        in_specs=[
            pl.BlockSpec((1024, DD), lambda i: (i, 0)),
            pl.BlockSpec((1024, DD), lambda i: (i, 0)),
            pl.BlockSpec((1024, DD), lambda i: (i, 0)),
            pl.BlockSpec((1024, DD), lambda i: (i, 0)),
            blk((NT, 2, DD)), blk((1, DD)), blk((1, DD)),
            blk((1, DD)), blk((1, DD)), blk((DD, DD)), blk((1, DD)),
        ],
        out_specs=[pl.BlockSpec((1024, DD), lambda i: (i, 0)),
                   pl.BlockSpec((2, DD), lambda i: (0, 0))],
        out_shape=[jax.ShapeDtypeStruct((NP, DD), _f32),
                   jax.ShapeDtypeStruct((2, DD), _f32)],
    )(xr, gc, gh, e, st, x0, c0, g, b, wv, vb)


def _k4_body(h_ref, sig_ref, cn_ref, wu_ref, ub_ref, hout_ref, enew_ref):
    cn = cn_ref[...]
    r = 1.0 / (cn[0] + EPS)
    enew_ref[...] = sig_ref[...] * r[None, :]
    hu = (lax.dot_general(h_ref[...], wu_ref[...], _DN_T,
                          preferred_element_type=_f32) + ub_ref[...])
    hout_ref[...] = jnp.maximum(hu + (cn[1] * r)[None, :], 0.0)


def _k4(h, sig, cn, wu, ub):
    return pl.pallas_call(
        _k4_body,
        grid=(NN // 2000,),
        in_specs=[
            pl.BlockSpec((2000, DD), lambda i: (i, 0)),
            pl.BlockSpec((2000, DD), lambda i: (i, 0)),
            pl.BlockSpec((2, DD), lambda i: (0, 0)),
            pl.BlockSpec((DD, DD), lambda i: (0, 0)),
            pl.BlockSpec((1, DD), lambda i: (0, 0)),
        ],
        out_specs=[pl.BlockSpec((2000, DD), lambda i: (i, 0))] * 2,
        out_shape=[jax.ShapeDtypeStruct((NN, DD), _f32)] * 2,
    )(h, sig, cn, wu, ub)


def kernel(h, e, edge_index, A_w, A_b, B_w, B_b, C_w, C_b, Dm_w, Dm_b,
           U_w, U_b, V_w, V_b, bn_g, bn_b):
    row = edge_index[0]
    col = edge_index[1]
    pad = jnp.zeros((NP - EE,), jnp.int32)
    rowp = jnp.concatenate([row, pad])
    colp = jnp.concatenate([col, pad])

    br = (A_b + Dm_b).reshape(1, DD)
    bc = (B_b + C_b).reshape(1, DD)
    vb = V_b.reshape(1, DD)
    ub = U_b.reshape(1, DD)

    cw, gh = _k0(rowp, colp, h)
    xr, xc = _k1(h, e, A_w, Dm_w, B_w, C_w, br, bc)
    st, gc = _k2(rowp, colp, cw, xr, xc)
    x0 = lax.slice(xr, (0, 0), (1, DD))
    c0 = lax.slice(xc, (0, 0), (1, DD))
    sig, cn = _k3(xr, gc, gh, e, st.reshape(NT, 2, DD), x0, c0,
                  bn_g.reshape(1, DD), bn_b.reshape(1, DD), V_w, vb)
    h_out, e_new = _k4(h, sig, cn, U_w, ub)
    return (h_out, e_new)


# speedup vs baseline: 4.4327x; 1.0189x over previous
import functools

import jax
import jax.numpy as jnp
from jax import lax
from jax.experimental import pallas as pl
from jax.experimental.pallas import tpu as pltpu
from jax.experimental.pallas import tpu_sc as plsc

NN = 100000
EE = 100000
DD = 128
NT = 32
PT = 3136
NP = NT * PT
CH = 224
NCH = PT // CH
CW_CH = 14336
EPAD = NP - EE
BN_EPS = 1e-5
EPS = 1e-5

_f32 = jnp.float32
_mesh = plsc.VectorSubcoreMesh(core_axis_name="c", subcore_axis_name="s")
_sc_params = pltpu.CompilerParams(needs_layout_passes=False)
_DN_T = (((1,), (1,)), ((), ()))


def _wid():
    return lax.axis_index("s") * 2 + lax.axis_index("c")


@functools.partial(
    pl.kernel,
    out_type=(jax.ShapeDtypeStruct((NP,), jnp.int32),
              jax.ShapeDtypeStruct((NP, DD), _f32)),
    mesh=_mesh,
    compiler_params=_sc_params,
    scratch_types=[
        pltpu.VMEM((PT,), jnp.int32),
        pltpu.VMEM((CW_CH,), jnp.int32),
        pltpu.VMEM((CW_CH,), jnp.int32),
        pltpu.VMEM((CH,), jnp.int32),
        pltpu.VMEM((CH,), jnp.int32),
        pltpu.VMEM((CH, DD), _f32),
        pltpu.VMEM((CH, DD), _f32),
        pltpu.SemaphoreType.DMA,
        pltpu.SemaphoreType.DMA,
    ],
)
def _k0(rowp_hbm, colp_hbm, h_hbm, cw_hbm, gh_hbm,
        slab, rbuf, cbuf, ci0, ci1, ba0, ba1, sem0, sem1):
    wid = _wid()
    base = wid * PT
    neg1 = jnp.full((16,), -1, jnp.int32)
    iota = lax.iota(jnp.int32, 16)

    sets = [(ci0, ba0, sem0), (ci1, ba1, sem1)]

    def issue_g(cix, S):
        ci, ba, sm = S
        noff = base + cix * CH
        pltpu.sync_copy(colp_hbm.at[pl.ds(noff, CH)], ci)
        pltpu.async_copy(h_hbm.at[ci], ba, sm)

    issue_g(0, sets[0])
    for cix in range(NCH):
        ci, ba, sm = sets[cix & 1]
        if cix + 1 < NCH:
            issue_g(cix + 1, sets[(cix + 1) & 1])
        pltpu.make_async_copy(h_hbm.at[ci], ba, sm).wait()
        pltpu.sync_copy(ba, gh_hbm.at[pl.ds(base + cix * CH, CH)])

    def init(i, _):
        slab[pl.ds(i * 16, 16)] = neg1
        return 0
    lax.fori_loop(0, PT // 16, init, 0)

    def chunk_w(cix, _):
        pltpu.sync_copy(rowp_hbm.at[pl.ds(cix * CW_CH, CW_CH)], rbuf)
        pltpu.sync_copy(colp_hbm.at[pl.ds(cix * CW_CH, CW_CH)], cbuf)
        kbase = cix * CW_CH

        def vreg(i, _):
            rv = rbuf[pl.ds(i * 16, 16)]
            cv = cbuf[pl.ds(i * 16, 16)]
            kvec = kbase + i * 16 + iota
            m = (rv >= base) & (rv < base + PT) & (kvec < EE)
            idx = jnp.clip(rv - base, 0, PT - 1)
            plsc.store_scatter(slab, [idx], cv, mask=m)
            return 0
        lax.fori_loop(0, CW_CH // 16, vreg, 0, unroll=4)
        return 0
    lax.fori_loop(0, NP // CW_CH, chunk_w, 0)
    pltpu.sync_copy(slab, cw_hbm.at[pl.ds(base, PT)])


def _k1_body(h_ref, e_ref, wa_ref, wd_ref, wb_ref, wcw_ref, br_ref, bc_ref,
             xr_ref, xc_ref):
    i = pl.program_id(0)
    rows = i * 1024 + lax.broadcasted_iota(jnp.int32, (1024, 1), 0)
    m = rows < NN
    hb = h_ref[...]
    eb = e_ref[...]
    xr_ref[...] = (lax.dot_general(hb, wa_ref[...], _DN_T, preferred_element_type=_f32)
                   + lax.dot_general(eb, wd_ref[...], _DN_T, preferred_element_type=_f32)
                   + br_ref[...])
    xc = (lax.dot_general(hb, wb_ref[...], _DN_T, preferred_element_type=_f32)
          + lax.dot_general(eb, wcw_ref[...], _DN_T, preferred_element_type=_f32)
          + bc_ref[...])
    xc_ref[...] = jnp.where(m, xc, -1e30)


def _k1(h, e, wa, wd, wb, wcw, br, bc):
    blk = lambda s: pl.BlockSpec(s, lambda i: (0,) * len(s))
    return pl.pallas_call(
        _k1_body,
        grid=(NP // 1024,),
        in_specs=[
            pl.BlockSpec((1024, DD), lambda i: (i, 0)),
            pl.BlockSpec((1024, DD), lambda i: (i, 0)),
            blk((DD, DD)), blk((DD, DD)), blk((DD, DD)), blk((DD, DD)),
            blk((1, DD)), blk((1, DD)),
        ],
        out_specs=[pl.BlockSpec((1024, DD), lambda i: (i, 0))] * 2,
        out_shape=[jax.ShapeDtypeStruct((NP, DD), _f32)] * 2,
    )(h, e, wa, wd, wb, wcw, br, bc)


@functools.partial(
    pl.kernel,
    out_type=(jax.ShapeDtypeStruct((NT * 2 * DD,), _f32),
              jax.ShapeDtypeStruct((NP, DD), _f32)),
    mesh=_mesh,
    compiler_params=_sc_params,
    scratch_types=[
        pltpu.VMEM((CH,), jnp.int32),
        pltpu.VMEM((CH,), jnp.int32),
        pltpu.VMEM((CH,), jnp.int32),
        pltpu.VMEM((CH,), jnp.int32),
        pltpu.VMEM((CH, DD), _f32),
        pltpu.VMEM((CH, DD), _f32),
        pltpu.VMEM((CH, DD), _f32),
        pltpu.VMEM((CH, DD), _f32),
        pltpu.VMEM((2 * DD,), _f32),
        pltpu.SemaphoreType.DMA,
        pltpu.SemaphoreType.DMA,
        pltpu.SemaphoreType.DMA,
        pltpu.SemaphoreType.DMA,
    ],
)
def _k2(rowp_hbm, colp_hbm, cw_hbm, xr_hbm, xc_hbm,
        st_hbm, gc_hbm,
        ri0, ci0, ri1, ci1, ba0, bb0, ba1, bb1, outb,
        sa0, sb0, sa1, sb1):
    wid = _wid()
    base = wid * PT
    zero = jnp.zeros((16,), _f32)
    iota = lax.iota(jnp.int32, 16)

    sets = [(ri0, ci0, ba0, bb0, sa0, sb0), (ri1, ci1, ba1, bb1, sa1, sb1)]

    def issue_a(cix, S):
        ri, ci, ba, bb, sa, sb = S
        off = base + cix * CH
        pltpu.sync_copy(rowp_hbm.at[pl.ds(off, CH)], ri)
        pltpu.sync_copy(colp_hbm.at[pl.ds(off, CH)], ci)
        pltpu.async_copy(xr_hbm.at[ri], ba, sa)
        pltpu.async_copy(xc_hbm.at[ci], bb, sb)

    issue_a(0, sets[0])
    carry = (tuple(zero for _ in range(8)), tuple(zero for _ in range(8)))
    for cix in range(NCH):
        ri, ci, ba, bb, sa, sb = sets[cix & 1]
        if cix + 1 < NCH:
            issue_a(cix + 1, sets[(cix + 1) & 1])
        pltpu.make_async_copy(xr_hbm.at[ri], ba, sa).wait()
        pltpu.make_async_copy(xc_hbm.at[ci], bb, sb).wait()

        def edge(i, car):
            s = list(car[0])
            q = list(car[1])
            for j in range(8):
                sl = pl.ds(j * 16, 16)
                t = ba[i, sl] + bb[i, sl]
                s[j] = s[j] + t
                q[j] = q[j] + t * t
            return (tuple(s), tuple(q))
        carry = lax.fori_loop(0, CH, edge, carry)
    s, q = carry
    for j in range(8):
        outb[pl.ds(j * 16, 16)] = s[j]
        outb[pl.ds(DD + j * 16, 16)] = q[j]
    pltpu.sync_copy(outb, st_hbm.at[pl.ds(wid * 2 * DD, 2 * DD)])

    def issue_b(cix, S):
        ri, ci, ba, bb, sa, sb = S
        noff = base + cix * CH
        pltpu.sync_copy(cw_hbm.at[pl.ds(noff, CH)], ri)

        def mk(i, _):
            cwv = ri[pl.ds(i * 16, 16)]
            spread = NN + ((i * 16 + iota) & 255)
            ci[pl.ds(i * 16, 16)] = jnp.where(cwv < 0, spread, cwv)
            return 0
        lax.fori_loop(0, CH // 16, mk, 0)
        pltpu.async_copy(xc_hbm.at[ci], ba, sa)

    issue_b(0, sets[0])
    for cix in range(NCH):
        ri, ci, ba, bb, sa, sb = sets[cix & 1]
        if cix + 1 < NCH:
            issue_b(cix + 1, sets[(cix + 1) & 1])
        pltpu.make_async_copy(xc_hbm.at[ci], ba, sa).wait()
        pltpu.sync_copy(ba, gc_hbm.at[pl.ds(base + cix * CH, CH)])


def _k3_body(xr_ref, gc_ref, gh_ref, e_ref, st_ref, x0_ref, c0_ref,
             g_ref, b_ref, wv_ref, vb_ref, sig_ref, cn_ref):
    i = pl.program_id(0)
    rows = i * 1024 + lax.broadcasted_iota(jnp.int32, (1024, 1), 0)
    stf = st_ref[...]
    ssum = jnp.sum(stf[:, 0, :], axis=0)
    qsum = jnp.sum(stf[:, 1, :], axis=0)
    t0 = (x0_ref[...] + c0_ref[...])[0]
    ssum = ssum - float(EPAD) * t0
    qsum = qsum - float(EPAD) * (t0 * t0)
    mean = ssum * (1.0 / EE)
    var = jnp.maximum(qsum * (1.0 / EE) - mean * mean, 0.0)
    s = g_ref[...][0] * lax.rsqrt(var + BN_EPS)
    c = b_ref[...][0] - mean * s
    t = xr_ref[...] + gc_ref[...]
    add = jnp.maximum(t * s[None, :] + c[None, :], 0.0)
    pre = jnp.where(rows < NN, e_ref[...] + add, -1e30)
    sig = jax.nn.sigmoid(pre)
    sig_ref[...] = sig
    gv = (lax.dot_general(gh_ref[...], wv_ref[...], _DN_T,
                          preferred_element_type=_f32) + vb_ref[...])
    ps = jnp.sum(sig, axis=0)
    pn = jnp.sum(sig * gv, axis=0)
    acc = jnp.stack([ps, pn], axis=0)

    @pl.when(i == 0)
    def _():
        cn_ref[...] = acc

    @pl.when(i > 0)
    def _():
        cn_ref[...] += acc


def _k3(xr, gc, gh, e, st, x0, c0, g, b, wv, vb):
    blk = lambda s: pl.BlockSpec(s, lambda i: (0,) * len(s))
    return pl.pallas_call(
        _k3_body,
        grid=(NP // 1024,),
        in_specs=[
            pl.BlockSpec((1024, DD), lambda i: (i, 0)),
            pl.BlockSpec((1024, DD), lambda i: (i, 0)),
            pl.BlockSpec((1024, DD), lambda i: (i, 0)),
            pl.BlockSpec((1024, DD), lambda i: (i, 0)),
            blk((NT, 2, DD)), blk((1, DD)), blk((1, DD)),
            blk((1, DD)), blk((1, DD)), blk((DD, DD)), blk((1, DD)),
        ],
        out_specs=[pl.BlockSpec((1024, DD), lambda i: (i, 0)),
                   pl.BlockSpec((2, DD), lambda i: (0, 0))],
        out_shape=[jax.ShapeDtypeStruct((NP, DD), _f32),
                   jax.ShapeDtypeStruct((2, DD), _f32)],
    )(xr, gc, gh, e, st, x0, c0, g, b, wv, vb)


def _k4_body(h_ref, sig_ref, cn_ref, wu_ref, ub_ref, hout_ref, enew_ref):
    cn = cn_ref[...]
    r = 1.0 / (cn[0] + EPS)
    enew_ref[...] = sig_ref[...] * r[None, :]
    hu = (lax.dot_general(h_ref[...], wu_ref[...], _DN_T,
                          preferred_element_type=_f32) + ub_ref[...])
    hout_ref[...] = jnp.maximum(hu + (cn[1] * r)[None, :], 0.0)


def _k4(h, sig, cn, wu, ub):
    return pl.pallas_call(
        _k4_body,
        grid=(NN // 2000,),
        in_specs=[
            pl.BlockSpec((2000, DD), lambda i: (i, 0)),
            pl.BlockSpec((2000, DD), lambda i: (i, 0)),
            pl.BlockSpec((2, DD), lambda i: (0, 0)),
            pl.BlockSpec((DD, DD), lambda i: (0, 0)),
            pl.BlockSpec((1, DD), lambda i: (0, 0)),
        ],
        out_specs=[pl.BlockSpec((2000, DD), lambda i: (i, 0))] * 2,
        out_shape=[jax.ShapeDtypeStruct((NN, DD), _f32)] * 2,
    )(h, sig, cn, wu, ub)


def kernel(h, e, edge_index, A_w, A_b, B_w, B_b, C_w, C_b, Dm_w, Dm_b,
           U_w, U_b, V_w, V_b, bn_g, bn_b):
    row = edge_index[0]
    col = edge_index[1]
    pad = jnp.zeros((NP - EE,), jnp.int32)
    rowp = jnp.concatenate([row, pad])
    colp = jnp.concatenate([col, pad])

    br = (A_b + Dm_b).reshape(1, DD)
    bc = (B_b + C_b).reshape(1, DD)
    vb = V_b.reshape(1, DD)
    ub = U_b.reshape(1, DD)

    cw, gh = _k0(rowp, colp, h)
    xr, xc = _k1(h, e, A_w, Dm_w, B_w, C_w, br, bc)
    st, gc = _k2(rowp, colp, cw, xr, xc)
    x0 = lax.slice(xr, (0, 0), (1, DD))
    c0 = lax.slice(xc, (0, 0), (1, DD))
    sig, cn = _k3(xr, gc, gh, e, st.reshape(NT, 2, DD), x0, c0,
                  bn_g.reshape(1, DD), bn_b.reshape(1, DD), V_w, vb)
    h_out, e_new = _k4(h, sig, cn, U_w, ub)
    return (h_out, e_new)
